# baseline (speedup 1.0000x reference)
import functools

import jax
import jax.numpy as jnp
from jax import lax
from jax.experimental import pallas as pl
from jax.experimental.pallas import tpu as pltpu
from jax.experimental.pallas import tpu_sc as plsc

_BATCH = 16384
_FIELDS = 26
_D = 64
_B = _BATCH * _FIELDS
_L = 128
_ROWS = _B // _L
_NC = 2
_NS = 16
_NW = _NC * _NS
_RPW = _ROWS // _NW
_CH = 4
_NCHUNK = _RPW // _CH


def _make_gather():
  mesh = plsc.VectorSubcoreMesh(core_axis_name="c", subcore_axis_name="s")

  @functools.partial(
      pl.kernel,
      out_type=jax.ShapeDtypeStruct((_B, _D), jnp.float32),
      mesh=mesh,
      compiler_params=pltpu.CompilerParams(use_tc_tiling_on_sc=False),
      scratch_types=[
          pltpu.VMEM((_CH, _L), jnp.int32),
          pltpu.VMEM((_CH * _L, _D), jnp.float32),
          pltpu.SemaphoreType.DMA,
      ],
  )
  def gather_kernel(idx_hbm, table_hbm, out_hbm, idx_v, rows_v, sem):
    wid = lax.axis_index("s") * _NC + lax.axis_index("c")
    base_row = wid * _RPW

    def body(c, carry):
      r0 = base_row + c * _CH
      pltpu.sync_copy(idx_hbm.at[pl.ds(r0, _CH)], idx_v)
      copies = [
          pltpu.async_copy(
              table_hbm.at[idx_v.at[j]],
              rows_v.at[pl.ds(j * _L, _L)],
              sem,
          )
          for j in range(_CH)
      ]
      for cp in copies:
        cp.wait()
      pltpu.sync_copy(rows_v, out_hbm.at[pl.ds(r0 * _L, _CH * _L)])
      return carry

    lax.fori_loop(0, _NCHUNK, body, 0)

  return gather_kernel


_gather = _make_gather()


@jax.jit
def kernel(concept_indices, table):
  idx2d = concept_indices.reshape(_ROWS, _L)
  out = _gather(idx2d, table)
  return out.reshape(_BATCH, _FIELDS, _D)


# speedup vs baseline: 1.0281x; 1.0281x over previous
import functools

import jax
import jax.numpy as jnp
from jax import lax
from jax.experimental import pallas as pl
from jax.experimental.pallas import tpu as pltpu
from jax.experimental.pallas import tpu_sc as plsc

_BATCH = 16384
_FIELDS = 26
_D = 64
_B = _BATCH * _FIELDS
_L = 128
_ROWS = _B // _L
_NC = 2
_NS = 16
_NW = _NC * _NS
_RPW = _ROWS // _NW
_CH = 4
_NCHUNK = _RPW // _CH


def _make_gather():
  mesh = plsc.VectorSubcoreMesh(core_axis_name="c", subcore_axis_name="s")

  @functools.partial(
      pl.kernel,
      out_type=jax.ShapeDtypeStruct((_B, _D), jnp.float32),
      mesh=mesh,
      compiler_params=pltpu.CompilerParams(use_tc_tiling_on_sc=False),
      scratch_types=[
          pltpu.VMEM((_RPW, _L), jnp.int32),
          pltpu.VMEM((_CH * _L, _D), jnp.float32),
          pltpu.VMEM((_CH * _L, _D), jnp.float32),
          pltpu.SemaphoreType.DMA,
          pltpu.SemaphoreType.DMA,
          pltpu.SemaphoreType.DMA,
          pltpu.SemaphoreType.DMA,
      ],
  )
  def gather_kernel(idx_hbm, table_hbm, out_hbm, idx_v, rows_v0, rows_v1,
                    sem_g0, sem_g1, sem_s0, sem_s1):
    wid = lax.axis_index("s") * _NC + lax.axis_index("c")
    base_row = wid * _RPW
    rows = (rows_v0, rows_v1)
    sem_g = (sem_g0, sem_g1)
    sem_s = (sem_s0, sem_s1)

    pltpu.sync_copy(idx_hbm.at[pl.ds(base_row, _RPW)], idx_v)

    def fire_gathers(c, b):
      for j in range(_CH):
        pltpu.async_copy(
            table_hbm.at[idx_v.at[c * _CH + j]],
            rows[b].at[pl.ds(j * _L, _L)],
            sem_g[b],
        )

    def wait_gathers(b):
      for j in range(_CH):
        pltpu.make_async_copy(
            table_hbm.at[idx_v.at[j]],
            rows[b].at[pl.ds(j * _L, _L)],
            sem_g[b],
        ).wait()

    def fire_store(c, b):
      pltpu.async_copy(
          rows[b],
          out_hbm.at[pl.ds((base_row + c * _CH) * _L, _CH * _L)],
          sem_s[b],
      )

    def wait_store(b):
      pltpu.make_async_copy(
          rows[b],
          out_hbm.at[pl.ds(base_row * _L, _CH * _L)],
          sem_s[b],
      ).wait()

    fire_gathers(0, 0)

    @pl.loop(0, _NCHUNK, step=2)
    def _outer(c0):
      for b in range(2):
        c = c0 + b
        nxt = c + 1

        @pl.when(nxt < _NCHUNK)
        def _():
          @pl.when(nxt >= 2)
          def _():
            wait_store(1 - b)
          fire_gathers(nxt, 1 - b)

        wait_gathers(b)
        fire_store(c, b)

    wait_store(0)
    wait_store(1)

  return gather_kernel


_gather = _make_gather()


@jax.jit
def kernel(concept_indices, table):
  idx2d = concept_indices.reshape(_ROWS, _L)
  out = _gather(idx2d, table)
  return out.reshape(_BATCH, _FIELDS, _D)
